# Initial kernel scaffold; baseline (speedup 1.0000x reference)
import jax
import jax.numpy as jnp
from jax.experimental import pallas as pl


def kernel(x, edge_index, edge_weight, W, b):
    raise NotImplementedError("write your pallas kernel here")


# speedup vs baseline: 9.3713x; 9.3713x over previous
import functools

import jax
import jax.numpy as jnp
from jax import lax
from jax.experimental import pallas as pl
from jax.experimental.pallas import tpu as pltpu
from jax.experimental.pallas import tpu_sc as plsc

N_PAD = 10240
E_PAD = 327680
D = 128
CH = 128
NC, NS = 2, 16
NW = NC * NS
RC_DEG = E_PAD // NS // CH
RC_MAIN = E_PAD // NW // CH
EWW = 40
NSL = N_PAD // NS

_mesh = plsc.VectorSubcoreMesh(
    core_axis_name="c", subcore_axis_name="s", num_cores=NC, num_subcores=NS)


@functools.partial(
    pl.kernel,
    out_type=(
        jax.ShapeDtypeStruct((NC, N_PAD, D), jnp.float32),
        jax.ShapeDtypeStruct((N_PAD,), jnp.float32),
    ),
    mesh=_mesh,
    compiler_params=pltpu.CompilerParams(needs_layout_passes=False),
    scratch_types=[
        pltpu.VMEM((2 * RC_MAIN, CH), jnp.int32),
        pltpu.VMEM((EWW, CH), jnp.float32),
        pltpu.VMEM((CH, D), jnp.float32),
        pltpu.VMEM((CH,), jnp.float32),
        pltpu.VMEM((CH,), jnp.float32),
        pltpu.VMEM((NSL,), jnp.float32),
        pltpu.VMEM_SHARED((N_PAD, D), jnp.float32),
        pltpu.VMEM_SHARED((N_PAD,), jnp.float32),
        pltpu.SemaphoreType.DMA,
    ],
)
def _sc_gcn(h_hbm, row_hbm, col_hbm, ew_hbm, outp_hbm, dis_hbm,
            idxbuf, ewm, rbuf, nbuf, cbuf, slicebuf, out_sp, deg_sp, sem):
    c = lax.axis_index("c")
    s = lax.axis_index("s")
    w = s * NC + c
    zero16 = jnp.zeros((16,), jnp.float32)

    def _zrow(i, _):
        for k in range(D // 16):
            rbuf[i, pl.ds(k * 16, 16)] = zero16
        return 0
    lax.fori_loop(0, CH, _zrow, 0)

    def _zslice(i, _):
        slicebuf[pl.ds(i * 16, 16)] = zero16
        return 0
    lax.fori_loop(0, NSL // 16, _zslice, 0)

    nb = s * NSL
    for k in range(NSL // CH):
        pltpu.sync_copy(rbuf, out_sp.at[pl.ds(nb + k * CH, CH)])
    pltpu.sync_copy(slicebuf, deg_sp.at[pl.ds(nb, NSL)])

    db = s * RC_DEG
    pltpu.sync_copy(col_hbm.at[pl.ds(db, RC_DEG)], idxbuf)
    plsc.subcore_barrier()

    for win in range(RC_DEG // EWW):
        pltpu.sync_copy(ew_hbm.at[pl.ds(db + win * EWW, EWW)], ewm)

        def _deg(j, _):
            pltpu.sync_copy(ewm.at[j], deg_sp.at[idxbuf.at[win * EWW + j]],
                            add=True)
            return 0
        lax.fori_loop(0, EWW, _deg, 0)
    plsc.subcore_barrier()

    pltpu.sync_copy(deg_sp.at[pl.ds(nb, NSL)], slicebuf)

    def _rs(i, _):
        d16 = slicebuf[pl.ds(i * 16, 16)] + 1.0
        y = lax.bitcast_convert_type(
            jnp.int32(0x5F3759DF)
            - (lax.bitcast_convert_type(d16, jnp.int32) >> 1),
            jnp.float32)
        for _ in range(3):
            y = y * (1.5 - 0.5 * d16 * y * y)
        slicebuf[pl.ds(i * 16, 16)] = y
        return 0
    lax.fori_loop(0, NSL // 16, _rs, 0)
    pltpu.sync_copy(slicebuf, deg_sp.at[pl.ds(nb, NSL)])

    @pl.when(c == 0)
    def _():
        pltpu.sync_copy(slicebuf, dis_hbm.at[pl.ds(nb, NSL)])
    plsc.subcore_barrier()

    mb = w * RC_MAIN
    pltpu.sync_copy(row_hbm.at[pl.ds(mb, RC_MAIN)], idxbuf.at[pl.ds(0, RC_MAIN)])
    pltpu.sync_copy(col_hbm.at[pl.ds(mb, RC_MAIN)],
                    idxbuf.at[pl.ds(RC_MAIN, RC_MAIN)])

    for win in range(RC_MAIN // EWW):
        pltpu.sync_copy(ew_hbm.at[pl.ds(mb + win * EWW, EWW)], ewm)

        def _chunk(jj, _):
            j = win * EWW + jj
            pltpu.async_copy(h_hbm.at[idxbuf.at[j]], rbuf, sem).wait()
            pltpu.sync_copy(deg_sp.at[idxbuf.at[j]], nbuf)
            pltpu.sync_copy(deg_sp.at[idxbuf.at[RC_MAIN + j]], cbuf)
            for k in range(CH // 16):
                sl = pl.ds(k * 16, 16)
                nbuf[sl] = nbuf[sl] * ewm[jj, sl] * cbuf[sl]

            def _scale(i, _):
                nsp = plsc.load_gather(nbuf, [jnp.broadcast_to(i, (16,))])
                for k in range(D // 16):
                    rbuf[i, pl.ds(k * 16, 16)] = rbuf[i, pl.ds(k * 16, 16)] * nsp
                return 0
            lax.fori_loop(0, CH, _scale, 0)
            pltpu.sync_copy(rbuf, out_sp.at[idxbuf.at[RC_MAIN + j]], add=True)
            return 0
        lax.fori_loop(0, EWW, _chunk, 0)
    plsc.subcore_barrier()

    pltpu.sync_copy(out_sp.at[pl.ds(nb, NSL)], outp_hbm.at[c, pl.ds(nb, NSL)])


def _mm_body(x_ref, w_ref, o_ref):
    o_ref[...] = jnp.dot(x_ref[...], w_ref[...],
                         preferred_element_type=jnp.float32)


_mm = pl.pallas_call(
    _mm_body,
    grid=(N_PAD // 1024,),
    in_specs=[pl.BlockSpec((1024, D), lambda i: (i, 0)),
              pl.BlockSpec((D, D), lambda i: (0, 0))],
    out_specs=pl.BlockSpec((1024, D), lambda i: (i, 0)),
    out_shape=jax.ShapeDtypeStruct((N_PAD, D), jnp.float32),
)


def _fin_body(p0_ref, p1_ref, dis_ref, h_ref, b_ref, o_ref):
    d = dis_ref[...]
    o_ref[...] = p0_ref[...] + p1_ref[...] + d * d * h_ref[...] + b_ref[...]


_fin = pl.pallas_call(
    _fin_body,
    grid=(N_PAD // 1024,),
    in_specs=[pl.BlockSpec((1024, D), lambda i: (i, 0)),
              pl.BlockSpec((1024, D), lambda i: (i, 0)),
              pl.BlockSpec((1024, 1), lambda i: (i, 0)),
              pl.BlockSpec((1024, D), lambda i: (i, 0)),
              pl.BlockSpec((1, D), lambda i: (0, 0))],
    out_specs=pl.BlockSpec((1024, D), lambda i: (i, 0)),
    out_shape=jax.ShapeDtypeStruct((N_PAD, D), jnp.float32),
)


def kernel(x, edge_index, edge_weight, W, b):
    N = x.shape[0]
    E = edge_weight.shape[0]
    row = edge_index[0].astype(jnp.int32)
    col = edge_index[1].astype(jnp.int32)
    rowp = jnp.concatenate(
        [row, jnp.zeros((E_PAD - E,), jnp.int32)]).reshape(E_PAD // CH, CH)
    colp = jnp.concatenate(
        [col, jnp.zeros((E_PAD - E,), jnp.int32)]).reshape(E_PAD // CH, CH)
    ewp = jnp.concatenate(
        [edge_weight.astype(jnp.float32),
         jnp.zeros((E_PAD - E,), jnp.float32)]).reshape(E_PAD // CH, CH)
    xp = jnp.concatenate([x, jnp.zeros((N_PAD - N, D), x.dtype)])

    h = _mm(xp, W)
    outp, dis = _sc_gcn(h, rowp, colp, ewp)
    out = _fin(outp[0], outp[1], dis.reshape(N_PAD, 1), h, b.reshape(1, D))
    return out[:N]


# speedup vs baseline: 10.4040x; 1.1102x over previous
import functools

import jax
import jax.numpy as jnp
from jax import lax
from jax.experimental import pallas as pl
from jax.experimental.pallas import tpu as pltpu
from jax.experimental.pallas import tpu_sc as plsc

N_PAD = 10240
E_PAD = 327680
D = 128
CH = 128
NC, NS = 2, 16
NW = NC * NS
RC_DEG = E_PAD // NS // CH
RC_MAIN = E_PAD // NW // CH
WIN = 8
NSL = N_PAD // NS

_mesh = plsc.VectorSubcoreMesh(
    core_axis_name="c", subcore_axis_name="s", num_cores=NC, num_subcores=NS)


@functools.partial(
    pl.kernel,
    out_type=(
        jax.ShapeDtypeStruct((NC, N_PAD, D), jnp.float32),
        jax.ShapeDtypeStruct((N_PAD,), jnp.float32),
    ),
    mesh=_mesh,
    compiler_params=pltpu.CompilerParams(needs_layout_passes=False),
    scratch_types=[
        pltpu.VMEM((2 * WIN, CH), jnp.int32),
        pltpu.VMEM((WIN, CH), jnp.float32),
        pltpu.VMEM((2, CH, D), jnp.float32),
        pltpu.VMEM((N_PAD,), jnp.float32),
        pltpu.VMEM((NSL,), jnp.float32),
        pltpu.VMEM_SHARED((N_PAD, D), jnp.float32),
        pltpu.VMEM_SHARED((N_PAD,), jnp.float32),
        pltpu.SemaphoreType.DMA,
        pltpu.SemaphoreType.DMA,
    ],
)
def _sc_gcn(h_hbm, row_hbm, col_hbm, ew_hbm, outp_hbm, dis_hbm,
            idxwin, ewwin, rbuf, disbuf, slicebuf, out_sp, deg_sp,
            dsem, gsem):
    c = lax.axis_index("c")
    s = lax.axis_index("s")
    w = s * NC + c
    zero16 = jnp.zeros((16,), jnp.float32)

    def _zrow(i, _):
        for k in range(D // 16):
            rbuf[0, i, pl.ds(k * 16, 16)] = zero16
        return 0
    lax.fori_loop(0, CH, _zrow, 0)

    def _zslice(i, _):
        slicebuf[pl.ds(i * 16, 16)] = zero16
        return 0
    lax.fori_loop(0, NSL // 16, _zslice, 0)

    nb = s * NSL
    for k in range(NSL // CH):
        pltpu.sync_copy(rbuf.at[0], out_sp.at[pl.ds(nb + k * CH, CH)])
    pltpu.sync_copy(slicebuf, deg_sp.at[pl.ds(nb, NSL)])
    plsc.subcore_barrier()

    db = s * RC_DEG
    for win in range(RC_DEG // WIN):
        wb = db + win * WIN
        pltpu.sync_copy(col_hbm.at[pl.ds(wb, WIN)], idxwin.at[pl.ds(0, WIN)])
        pltpu.sync_copy(ew_hbm.at[pl.ds(wb, WIN)], ewwin)
        descs = [
            pltpu.async_copy(ewwin.at[j], deg_sp.at[idxwin.at[j]], dsem,
                             add=True)
            for j in range(WIN)
        ]
        for dsc in descs:
            dsc.wait()
    plsc.subcore_barrier()

    pltpu.sync_copy(deg_sp.at[pl.ds(nb, NSL)], slicebuf)

    def _rs(i, _):
        d16 = slicebuf[pl.ds(i * 16, 16)] + 1.0
        y = lax.bitcast_convert_type(
            jnp.int32(0x5F3759DF)
            - (lax.bitcast_convert_type(d16, jnp.int32) >> 1),
            jnp.float32)
        for _ in range(3):
            y = y * (1.5 - 0.5 * d16 * y * y)
        slicebuf[pl.ds(i * 16, 16)] = y
        return 0
    lax.fori_loop(0, NSL // 16, _rs, 0)
    pltpu.sync_copy(slicebuf, deg_sp.at[pl.ds(nb, NSL)])

    @pl.when(c == 0)
    def _():
        pltpu.sync_copy(slicebuf, dis_hbm.at[pl.ds(nb, NSL)])
    plsc.subcore_barrier()
    pltpu.sync_copy(deg_sp, disbuf)

    def _norm(jj):
        for k in range(CH // 16):
            sl = pl.ds(k * 16, 16)
            r16 = idxwin[jj, sl]
            c16 = idxwin[WIN + jj, sl]
            ewwin[jj, sl] = (plsc.load_gather(disbuf, [r16]) * ewwin[jj, sl]
                             * plsc.load_gather(disbuf, [c16]))

    def _scale_scatter(jj, p):
        def _scale(i, _):
            for r in range(2):
                i2 = i * 2 + r
                nsp = plsc.load_gather(ewwin.at[jj],
                                       [jnp.broadcast_to(i2, (16,))])
                for k in range(D // 16):
                    rbuf[p, i2, pl.ds(k * 16, 16)] = (
                        rbuf[p, i2, pl.ds(k * 16, 16)] * nsp)
            return 0
        lax.fori_loop(0, CH // 2, _scale, 0)
        pltpu.sync_copy(rbuf.at[p], out_sp.at[idxwin.at[WIN + jj]], add=True)

    mb = w * RC_MAIN
    for win in range(RC_MAIN // WIN):
        wb = mb + win * WIN
        pltpu.sync_copy(row_hbm.at[pl.ds(wb, WIN)], idxwin.at[pl.ds(0, WIN)])
        pltpu.sync_copy(col_hbm.at[pl.ds(wb, WIN)],
                        idxwin.at[pl.ds(WIN, WIN)])
        pltpu.sync_copy(ew_hbm.at[pl.ds(wb, WIN)], ewwin)

        def _pair(t, _):
            a = 2 * t
            da = pltpu.async_copy(h_hbm.at[idxwin.at[a]], rbuf.at[0], gsem)
            _norm(a)
            _norm(a + 1)
            da.wait()
            db = pltpu.async_copy(h_hbm.at[idxwin.at[a + 1]], rbuf.at[1],
                                  gsem)
            _scale_scatter(a, 0)
            db.wait()
            _scale_scatter(a + 1, 1)
            return 0
        lax.fori_loop(0, WIN // 2, _pair, 0)
    plsc.subcore_barrier()

    pltpu.sync_copy(out_sp.at[pl.ds(nb, NSL)], outp_hbm.at[c, pl.ds(nb, NSL)])


def _mm_body(x_ref, w_ref, o_ref):
    o_ref[...] = jnp.dot(x_ref[...], w_ref[...],
                         preferred_element_type=jnp.float32)


_mm = pl.pallas_call(
    _mm_body,
    grid=(N_PAD // 1024,),
    in_specs=[pl.BlockSpec((1024, D), lambda i: (i, 0)),
              pl.BlockSpec((D, D), lambda i: (0, 0))],
    out_specs=pl.BlockSpec((1024, D), lambda i: (i, 0)),
    out_shape=jax.ShapeDtypeStruct((N_PAD, D), jnp.float32),
)


def _fin_body(p0_ref, p1_ref, dis_ref, h_ref, b_ref, o_ref):
    d = dis_ref[...]
    o_ref[...] = p0_ref[...] + p1_ref[...] + d * d * h_ref[...] + b_ref[...]


_fin = pl.pallas_call(
    _fin_body,
    grid=(N_PAD // 1024,),
    in_specs=[pl.BlockSpec((1024, D), lambda i: (i, 0)),
              pl.BlockSpec((1024, D), lambda i: (i, 0)),
              pl.BlockSpec((1024, 1), lambda i: (i, 0)),
              pl.BlockSpec((1024, D), lambda i: (i, 0)),
              pl.BlockSpec((1, D), lambda i: (0, 0))],
    out_specs=pl.BlockSpec((1024, D), lambda i: (i, 0)),
    out_shape=jax.ShapeDtypeStruct((N_PAD, D), jnp.float32),
)


def kernel(x, edge_index, edge_weight, W, b):
    N = x.shape[0]
    E = edge_weight.shape[0]
    row = edge_index[0].astype(jnp.int32)
    col = edge_index[1].astype(jnp.int32)
    rowp = jnp.concatenate(
        [row, jnp.zeros((E_PAD - E,), jnp.int32)]).reshape(E_PAD // CH, CH)
    colp = jnp.concatenate(
        [col, jnp.zeros((E_PAD - E,), jnp.int32)]).reshape(E_PAD // CH, CH)
    ewp = jnp.concatenate(
        [edge_weight.astype(jnp.float32),
         jnp.zeros((E_PAD - E,), jnp.float32)]).reshape(E_PAD // CH, CH)
    xp = jnp.concatenate([x, jnp.zeros((N_PAD - N, D), x.dtype)])

    h = _mm(xp, W)
    outp, dis = _sc_gcn(h, rowp, colp, ewp)
    out = _fin(outp[0], outp[1], dis.reshape(N_PAD, 1), h, b.reshape(1, D))
    return out[:N]
